# baseline (speedup 1.0000x reference)
import functools

import numpy as np
import jax
import jax.numpy as jnp
from jax import lax
from jax.experimental import pallas as pl
from jax.experimental.pallas import tpu as pltpu
from jax.experimental.pallas import tpu_sc as plsc

_VOCAB = 100000
_D = 128
_SEQ = 200
_BATCH = 1024
_NW = 32
_CHUNK = 100
_SPW = _BATCH // _NW
_NCH = _SPW * 2
_NBUF = 3
_SCALE = float(np.sqrt(float(_D)))


def _pos_table() -> np.ndarray:
    d = np.arange(_D)
    even = (d % 2 == 0).astype(np.float64)
    odd = (d % 2 == 1).astype(np.float64)
    rate = 1.0 / (10000.0 ** (d[np.newaxis, :] / _D))
    rads = np.arange(_SEQ)[:, np.newaxis] * rate
    return (np.sin(rads) * even + np.cos(rads) * odd).astype(np.float32)


def _pos_packed() -> np.ndarray:
    import ml_dtypes
    u = (_pos_table().astype(ml_dtypes.bfloat16)
         .view(np.uint16).astype(np.uint32))
    out = np.empty((_SEQ, _D // 2), np.uint32)
    for j2 in range(_D // 32):
        a = u[:, 32 * j2:32 * j2 + 16]
        b = u[:, 32 * j2 + 16:32 * j2 + 32]
        out[:, 16 * j2:16 * (j2 + 1)] = a | (b << 16)
    return out.view(np.int32)


_POS_PACKED = _pos_packed()

_mesh = plsc.VectorSubcoreMesh(core_axis_name="c", subcore_axis_name="s")


@functools.partial(
    pl.kernel,
    mesh=_mesh,
    out_type=jax.ShapeDtypeStruct((_BATCH, _SEQ, _D), jnp.float32),
    scratch_types=[
        pltpu.VMEM((_SPW * _SEQ,), jnp.int32),
        pltpu.VMEM((_SEQ, _D // 2), jnp.int32),
        pltpu.VMEM((_NBUF, _SEQ, _D), jnp.float32),
        pltpu.SemaphoreType.DMA,
        pltpu.SemaphoreType.DMA,
        pltpu.SemaphoreType.DMA,
        pltpu.SemaphoreType.DMA,
        pltpu.SemaphoreType.DMA,
        pltpu.SemaphoreType.DMA,
    ],
)
def _emb_lookup(idx_hbm, tab_hbm, pos_hbm, out_hbm, idx_v, pos_v, buf,
                gs0, gs1, gs2, ss0, ss1, ss2):
    gsems = (gs0, gs1, gs2)
    ssems = (ss0, ss1, ss2)
    wid = lax.axis_index("s") * 2 + lax.axis_index("c")
    b_base = wid * _SPW
    pltpu.sync_copy(pos_hbm, pos_v)
    pltpu.sync_copy(idx_hbm.at[wid], idx_v)

    def issue_gather(q, s):
        pltpu.async_copy(tab_hbm.at[idx_v.at[pl.ds(q * _SEQ, 128)]],
                         buf.at[s, pl.ds(0, 128)], gsems[s])
        pltpu.async_copy(tab_hbm.at[idx_v.at[pl.ds(q * _SEQ + 128, _SEQ - 128)]],
                         buf.at[s, pl.ds(128, _SEQ - 128)], gsems[s])

    def wait_gather(s):
        pltpu.make_async_copy(out_hbm.at[0], buf.at[s], gsems[s]).wait()

    def issue_store(q, s):
        pltpu.async_copy(buf.at[s], out_hbm.at[b_base + q], ssems[s])

    def wait_store(s):
        pltpu.make_async_copy(buf.at[s], out_hbm.at[0], ssems[s]).wait()

    def compute(s):
        def row_body(r2, c):
            for u in range(2):
                r = 2 * r2 + u
                for j2 in range(_D // 32):
                    w = pos_v[r, pl.ds(16 * j2, 16)]
                    pa = lax.bitcast_convert_type(w << 16, jnp.float32)
                    pb = lax.bitcast_convert_type(w & jnp.int32(-65536),
                                                  jnp.float32)
                    sl0 = pl.ds(32 * j2, 16)
                    sl1 = pl.ds(32 * j2 + 16, 16)
                    buf[s, r, sl0] = buf[s, r, sl0] * _SCALE + pa
                    buf[s, r, sl1] = buf[s, r, sl1] * _SCALE + pb
            return c

        lax.fori_loop(0, _SEQ // 2, row_body, 0)

    for q in range(_NBUF - 1):
        issue_gather(q, q)

    wait_gather(0)
    compute(0)
    issue_gather(2, 2)
    issue_store(0, 0)
    for q in range(1, _NBUF):
        s = q % _NBUF
        wait_gather(s)
        compute(s)
        wait_store((q - 1) % _NBUF)
        issue_gather(q + 2, (q + 2) % _NBUF)
        issue_store(q, s)

    def block_body(it, carry):
        q0 = it * _NBUF
        for b in range(_NBUF):
            q = q0 + b
            wait_gather(b)
            compute(b)
            wait_store((b - 1) % _NBUF)
            issue_gather(q + 2, (b + 2) % _NBUF)
            issue_store(q, b)
        return carry

    lax.fori_loop(1, _SPW // _NBUF, block_body, 0)

    for q in (_SPW - 2, _SPW - 1):
        s = q % _NBUF
        wait_gather(s)
        compute(s)
        wait_store((q - 1) % _NBUF)
        issue_store(q, s)
    wait_store((_SPW - 1) % _NBUF)


def kernel(x, embedding_table):
    idx = x.reshape(_NW, _SPW * _SEQ).astype(jnp.int32)
    pos = jnp.asarray(_POS_PACKED)
    return _emb_lookup(idx, embedding_table, pos)
